# baseline (speedup 1.0000x reference)
import jax
import jax.numpy as jnp
from jax.experimental import pallas as pl
from jax.experimental.pallas import tpu as pltpu

BLOCK = 1024
NRING = 2


def _gate_kernel(x_hbm, w_ref, o_ref, buf, sem):
    nblocks = x_hbm.shape[0] // BLOCK

    def copy(i, slot):
        return pltpu.make_async_copy(
            x_hbm.at[pl.ds(i * BLOCK, BLOCK), :],
            buf.at[slot],
            sem.at[slot],
        )

    for s in range(NRING):
        copy(s, s).start()

    w = w_ref[...]
    for i in range(nblocks):
        slot = i % NRING
        copy(i, slot).wait()
        logits_t = jax.lax.dot_general(
            w,
            buf[slot],
            dimension_numbers=(((1,), (1,)), ((), ())),
            preferred_element_type=jnp.float32,
            precision=jax.lax.Precision.DEFAULT,
        )
        if i + NRING < nblocks:
            copy(i + NRING, slot).start()
        o_ref[:, pl.ds(i * BLOCK, BLOCK)] = (logits_t > 0.0).astype(jnp.int32)


@jax.jit
def kernel(cls_hidden_states, gate_w):
    tokens, hidden = cls_hidden_states.shape
    num_experts = gate_w.shape[0]

    mask_t = pl.pallas_call(
        _gate_kernel,
        in_specs=[
            pl.BlockSpec(memory_space=pltpu.MemorySpace.HBM),
            pl.BlockSpec(memory_space=pltpu.MemorySpace.VMEM),
        ],
        out_specs=pl.BlockSpec(memory_space=pltpu.MemorySpace.VMEM),
        out_shape=jax.ShapeDtypeStruct((num_experts, tokens), jnp.int32),
        scratch_shapes=[
            pltpu.VMEM((NRING, BLOCK, hidden), jnp.float32),
            pltpu.SemaphoreType.DMA((NRING,)),
        ],
    )(cls_hidden_states, gate_w)
    return mask_t.T
